# Initial kernel scaffold; baseline (speedup 1.0000x reference)
import jax
import jax.numpy as jnp
from jax.experimental import pallas as pl


def kernel(attn_weights, class_poss, final_poss, weight):
    raise NotImplementedError("write your pallas kernel here")


# speedup vs baseline: 2.3218x; 2.3218x over previous
import jax
import jax.numpy as jnp
from jax import lax
from jax.experimental import pallas as pl
from jax.experimental.pallas import tpu as pltpu

SEQ = 2048
N_HEAD = 32
BR = 256
DEMO_PAD = 64


def _copy_fixup_body(final_ref, class_ref, attn_ref, w_ref, out_ref):
    rb = pl.program_id(1)
    out_ref[...] = attn_ref[...]
    base = rb * BR
    wrow = jnp.exp(w_ref[0, :, :])
    col = lax.broadcasted_iota(jnp.int32, (1, SEQ), 1)
    n_demo = final_ref.shape[0]
    for d in range(n_demo):
        f = final_ref[d]
        c = class_ref[d]

        @pl.when((f >= base) & (f < base + BR))
        def _(d=d, f=f, c=c):
            rl = f - base
            cur = out_ref[0, 0, pl.ds(rl, 1), :]
            orig = attn_ref[0, 0, pl.ds(rl, 1), :]
            wd = wrow[:, d:d + 1]
            out_ref[0, 0, pl.ds(rl, 1), :] = jnp.where(col == c, orig * wd, cur)


def kernel(attn_weights, class_poss, final_poss, weight):
    n_head = attn_weights.shape[1]
    seq = attn_weights.shape[2]
    wpad = jnp.zeros((n_head, 1, DEMO_PAD), jnp.float32)
    wpad = wpad.at[:, 0, : weight.shape[1]].set(weight)

    grid_spec = pltpu.PrefetchScalarGridSpec(
        num_scalar_prefetch=2,
        grid=(n_head, seq // BR),
        in_specs=[
            pl.BlockSpec((1, 1, BR, seq), lambda h, rb, *_: (0, h, rb, 0)),
            pl.BlockSpec((1, 1, DEMO_PAD), lambda h, rb, *_: (h, 0, 0)),
        ],
        out_specs=pl.BlockSpec((1, 1, BR, seq), lambda h, rb, *_: (0, h, rb, 0)),
    )
    return pl.pallas_call(
        _copy_fixup_body,
        grid_spec=grid_spec,
        out_shape=jax.ShapeDtypeStruct(attn_weights.shape, attn_weights.dtype),
    )(final_poss, class_poss, attn_weights, wpad)


# speedup vs baseline: 2.9397x; 1.2662x over previous
import jax
import jax.numpy as jnp
from jax import lax
from jax.experimental import pallas as pl
from jax.experimental.pallas import tpu as pltpu

SEQ = 2048
N_HEAD = 32
BR = 512
DEMO_PAD = 64


def _copy_fixup_body(final_ref, class_ref, attn_ref, w_ref, out_ref):
    rb = pl.program_id(1)
    out_ref[...] = attn_ref[...]
    base = rb * BR
    wrow = jnp.exp(w_ref[0, :, :])
    col = lax.broadcasted_iota(jnp.int32, (1, SEQ), 1)
    n_demo = final_ref.shape[0]
    for d in range(n_demo):
        f = final_ref[d]
        c = class_ref[d]

        @pl.when((f >= base) & (f < base + BR))
        def _(d=d, f=f, c=c):
            rl = f - base
            cur = out_ref[0, 0, pl.ds(rl, 1), :]
            orig = attn_ref[0, 0, pl.ds(rl, 1), :]
            wd = wrow[:, d:d + 1]
            out_ref[0, 0, pl.ds(rl, 1), :] = jnp.where(col == c, orig * wd, cur)


def kernel(attn_weights, class_poss, final_poss, weight):
    n_head = attn_weights.shape[1]
    seq = attn_weights.shape[2]
    wpad = jnp.zeros((n_head, 1, DEMO_PAD), jnp.float32)
    wpad = wpad.at[:, 0, : weight.shape[1]].set(weight)

    grid_spec = pltpu.PrefetchScalarGridSpec(
        num_scalar_prefetch=2,
        grid=(n_head, seq // BR),
        in_specs=[
            pl.BlockSpec((1, 1, BR, seq), lambda h, rb, *_: (0, h, rb, 0)),
            pl.BlockSpec((1, 1, DEMO_PAD), lambda h, rb, *_: (h, 0, 0)),
        ],
        out_specs=pl.BlockSpec((1, 1, BR, seq), lambda h, rb, *_: (0, h, rb, 0)),
    )
    return pl.pallas_call(
        _copy_fixup_body,
        grid_spec=grid_spec,
        out_shape=jax.ShapeDtypeStruct(attn_weights.shape, attn_weights.dtype),
    )(final_poss, class_poss, attn_weights, wpad)


# speedup vs baseline: 3.1206x; 1.0615x over previous
import jax
import jax.numpy as jnp
from jax import lax
from jax.experimental import pallas as pl
from jax.experimental.pallas import tpu as pltpu

SEQ = 2048
N_HEAD = 32
BR = 1024
DEMO_PAD = 64


def _copy_fixup_body(final_ref, class_ref, attn_ref, w_ref, out_ref):
    rb = pl.program_id(1)
    out_ref[...] = attn_ref[...]
    base = rb * BR
    wrow = jnp.exp(w_ref[0, :, :])
    col = lax.broadcasted_iota(jnp.int32, (1, SEQ), 1)
    n_demo = final_ref.shape[0]
    for d in range(n_demo):
        f = final_ref[d]
        c = class_ref[d]

        @pl.when((f >= base) & (f < base + BR))
        def _(d=d, f=f, c=c):
            rl = f - base
            cur = out_ref[0, 0, pl.ds(rl, 1), :]
            orig = attn_ref[0, 0, pl.ds(rl, 1), :]
            wd = wrow[:, d:d + 1]
            out_ref[0, 0, pl.ds(rl, 1), :] = jnp.where(col == c, orig * wd, cur)


def kernel(attn_weights, class_poss, final_poss, weight):
    n_head = attn_weights.shape[1]
    seq = attn_weights.shape[2]
    wpad = jnp.zeros((n_head, 1, DEMO_PAD), jnp.float32)
    wpad = wpad.at[:, 0, : weight.shape[1]].set(weight)

    grid_spec = pltpu.PrefetchScalarGridSpec(
        num_scalar_prefetch=2,
        grid=(n_head, seq // BR),
        in_specs=[
            pl.BlockSpec((1, 1, BR, seq), lambda h, rb, *_: (0, h, rb, 0)),
            pl.BlockSpec((1, 1, DEMO_PAD), lambda h, rb, *_: (h, 0, 0)),
        ],
        out_specs=pl.BlockSpec((1, 1, BR, seq), lambda h, rb, *_: (0, h, rb, 0)),
    )
    return pl.pallas_call(
        _copy_fixup_body,
        grid_spec=grid_spec,
        out_shape=jax.ShapeDtypeStruct(attn_weights.shape, attn_weights.dtype),
    )(final_poss, class_poss, attn_weights, wpad)
